# baseline (speedup 1.0000x reference)
import functools

import jax
import jax.numpy as jnp
from jax.experimental import pallas as pl
from jax.experimental.pallas import tpu as pltpu
from jax.experimental.pallas import tpu_sc as plsc

N = 10000
D_FEAT = 128
K = 16
EPS = 1e-8
LN_EPS = 1e-5
SLOPE = 0.1

E = N * K
E_BLK = 3200


def _lrelu(h):
    return jnp.where(h >= 0, h, SLOPE * h)


def _enc_block(raw_ref, W1_ref, b1_ref, g1_ref, be1_ref, W2_ref, b2_ref,
               g2_ref, be2_ref, out_ref):
    raw = raw_ref[...]
    h = jnp.dot(raw, W1_ref[...].T, preferred_element_type=jnp.float32)
    h = h + b1_ref[...]
    mu = jnp.mean(h, axis=-1, keepdims=True)
    var = jnp.mean((h - mu) ** 2, axis=-1, keepdims=True)
    h = (h - mu) / jnp.sqrt(var + LN_EPS) * g1_ref[...] + be1_ref[...]
    h = _lrelu(h)
    h = jnp.dot(h, W2_ref[...].T, preferred_element_type=jnp.float32)
    h = h + b2_ref[...]
    mu = jnp.mean(h, axis=-1, keepdims=True)
    var = jnp.mean((h - mu) ** 2, axis=-1, keepdims=True)
    h = (h - mu) / jnp.sqrt(var + LN_EPS) * g2_ref[...] + be2_ref[...]
    out_ref[...] = _lrelu(h)


def _encode(raw, W1, b1, g1, be1, W2, b2, g2, be2):
    grid = E // E_BLK
    return pl.pallas_call(
        _enc_block,
        grid=(grid,),
        in_specs=[
            pl.BlockSpec((E_BLK, 5), lambda i: (i, 0)),
            pl.BlockSpec((32, 5), lambda i: (0, 0)),
            pl.BlockSpec((1, 32), lambda i: (0, 0)),
            pl.BlockSpec((1, 32), lambda i: (0, 0)),
            pl.BlockSpec((1, 32), lambda i: (0, 0)),
            pl.BlockSpec((16, 32), lambda i: (0, 0)),
            pl.BlockSpec((1, 16), lambda i: (0, 0)),
            pl.BlockSpec((1, 16), lambda i: (0, 0)),
            pl.BlockSpec((1, 16), lambda i: (0, 0)),
        ],
        out_specs=pl.BlockSpec((E_BLK, 16), lambda i: (i, 0)),
        out_shape=jax.ShapeDtypeStruct((E, 16), jnp.float32),
    )(raw, W1, b1.reshape(1, 32), g1.reshape(1, 32), be1.reshape(1, 32),
      W2, b2.reshape(1, 16), g2.reshape(1, 16), be2.reshape(1, 16))


NPAD = 10240
R_BLK = 256
N_BLKS = 40


F_FOLD = 8
W_FOLD = NPAD // F_FOLD
INF = 3e38


def _rbf16(v):
    u = jax.lax.bitcast_convert_type(v, jnp.int32)
    u = u + 0x7FFF + ((u >> 16) & 1)
    u = (u >> 16) << 16
    return jax.lax.bitcast_convert_type(u, jnp.float32)


def _knn_block(posr0, posr1, sqr, posc0, posc1, sqc, out_ref):
    i = pl.program_id(0)
    r0 = _rbf16(posr0[...])
    r1 = _rbf16(posr1[...])
    rs = sqr[...]
    colj = jax.lax.broadcasted_iota(jnp.int32, (R_BLK, W_FOLD), 1)
    rowg = jax.lax.broadcasted_iota(jnp.int32, (R_BLK, W_FOLD), 0) + i * R_BLK

    M1 = jnp.full((R_BLK, W_FOLD), INF, jnp.float32)
    M2 = M1
    M3 = M1
    FI1 = jnp.zeros((R_BLK, W_FOLD), jnp.int32)
    FI2 = FI1
    FI3 = FI1
    for c in range(F_FOLD):
        c0 = _rbf16(posc0[c:c + 1, :])
        c1 = _rbf16(posc1[c:c + 1, :])
        cs = sqc[c:c + 1, :]
        d = rs + cs - 2.0 * (r0 * c0 + r1 * c1)
        d = jnp.where(colj * F_FOLD + c == rowg, d + 1e10, d)
        lt1 = d < M1
        lt2 = d < M2
        lt3 = d < M3
        ci = jnp.full((R_BLK, W_FOLD), c, jnp.int32)
        M3 = jnp.where(lt3, jnp.where(lt2, M2, d), M3)
        FI3 = jnp.where(lt3, jnp.where(lt2, FI2, ci), FI3)
        M2 = jnp.where(lt2, jnp.where(lt1, M1, d), M2)
        FI2 = jnp.where(lt2, jnp.where(lt1, FI1, ci), FI2)
        M1 = jnp.where(lt1, d, M1)
        FI1 = jnp.where(lt1, ci, FI1)

    tcol = jax.lax.broadcasted_iota(jnp.int32, (R_BLK, K), 1)
    nbr = jnp.zeros((R_BLK, K), jnp.int32)
    for t in range(K):
        m = jnp.min(M1, axis=1, keepdims=True)
        hit = M1 == m
        jj = jnp.min(jnp.where(hit, colj, NPAD), axis=1, keepdims=True)
        upd = colj == jj
        fi = jnp.min(jnp.where(upd, FI1, F_FOLD), axis=1, keepdims=True)
        nbr = jnp.where(tcol == t, jj * F_FOLD + fi, nbr)
        M1 = jnp.where(upd, M2, M1)
        FI1 = jnp.where(upd, FI2, FI1)
        M2 = jnp.where(upd, M3, M2)
        FI2 = jnp.where(upd, FI3, FI2)
        M3 = jnp.where(upd, INF, M3)
    out_ref[...] = nbr


def _knn_topk(geometric_info):
    pos0 = geometric_info[:, 6]
    pos1 = geometric_info[:, 7]
    sq = pos0 * pos0 + pos1 * pos1
    pad = NPAD - N
    pos0f = jnp.pad(pos0, (0, pad)).reshape(W_FOLD, F_FOLD).T
    pos1f = jnp.pad(pos1, (0, pad)).reshape(W_FOLD, F_FOLD).T
    sqf = jnp.pad(sq, (0, pad), constant_values=3e38).reshape(W_FOLD, F_FOLD).T
    rpad = N_BLKS * R_BLK - N
    row0 = jnp.pad(pos0, (0, rpad)).reshape(-1, 1)
    row1 = jnp.pad(pos1, (0, rpad)).reshape(-1, 1)
    rowsq = jnp.pad(sq, (0, rpad)).reshape(-1, 1)
    nbr = pl.pallas_call(
        _knn_block,
        grid=(N_BLKS,),
        in_specs=[
            pl.BlockSpec((R_BLK, 1), lambda i: (i, 0)),
            pl.BlockSpec((R_BLK, 1), lambda i: (i, 0)),
            pl.BlockSpec((R_BLK, 1), lambda i: (i, 0)),
            pl.BlockSpec((F_FOLD, W_FOLD), lambda i: (0, 0)),
            pl.BlockSpec((F_FOLD, W_FOLD), lambda i: (0, 0)),
            pl.BlockSpec((F_FOLD, W_FOLD), lambda i: (0, 0)),
        ],
        out_specs=pl.BlockSpec((R_BLK, K), lambda i: (i, 0)),
        out_shape=jax.ShapeDtypeStruct((N_BLKS * R_BLK, K), jnp.int32),
    )(row0, row1, rowsq, pos0f, pos1f, sqf)
    return nbr[:N]


NW = 32
TPW = 320
NTPAD = NW * TPW
EW = TPW * K


def _norm_block(x_ref, o_ref):
    xv = x_ref[...]
    o_ref[...] = jnp.sqrt(jnp.sum(xv * xv, axis=1, keepdims=True))


def _x_norms(x):
    out = pl.pallas_call(
        _norm_block,
        grid=(25,),
        in_specs=[pl.BlockSpec((400, D_FEAT), lambda i: (i, 0))],
        out_specs=pl.BlockSpec((400, 1), lambda i: (i, 0)),
        out_shape=jax.ShapeDtypeStruct((N, 1), jnp.float32),
    )(x)
    return out.reshape(N)


def _edge_sc_body(x_hbm, geo_hbm, c2_hbm, c3_hbm, c6_hbm, c7_hbm, nbr_hbm,
                  out_hbm,
                  nbrs_v, gt0_v, gt1_v, c2b0, c2b1, c3b0, c3b1, c6b0, c6b1,
                  c7b0, c7b1, xt0_v, xt1_v, xs0_v, xs1_v, obuf_v,
                  sg0, sg1, sm0, sm1):
    cid = jax.lax.axis_index("c")
    sid = jax.lax.axis_index("s")
    wid = sid * 2 + cid
    base = wid * TPW
    pltpu.sync_copy(nbr_hbm.at[pl.ds(base, TPW)], nbrs_v)
    lanes = jax.lax.iota(jnp.int32, 16)
    xs_v = (xs0_v, xs1_v)
    xt_v = (xt0_v, xt1_v)
    gt_v = (gt0_v, gt1_v)
    cols_v = ((c2b0, c3b0, c6b0, c7b0), (c2b1, c3b1, c6b1, c7b1))
    cols_hbm = (c2_hbm, c3_hbm, c6_hbm, c7_hbm)
    sg = (sg0, sg1)
    sm = (sm0, sm1)

    def issue(t, b):
        tc = jnp.minimum(t, TPW - 1)
        i_safe = jnp.minimum(base + tc, N - 1)
        idx = nbrs_v[tc]
        pltpu.async_copy(x_hbm.at[idx], xs_v[b], sg[b])
        pltpu.async_copy(x_hbm.at[i_safe], xt_v[b], sm[b])
        pltpu.async_copy(geo_hbm.at[i_safe], gt_v[b], sm[b])
        for q in range(4):
            pltpu.async_copy(cols_hbm[q].at[idx], cols_v[b][q], sm[b])

    issue(0, 0)
    issue(1, 1)

    def drain(b):
        pltpu.make_async_copy(x_hbm.at[pl.ds(0, 16)], xs_v[b], sg[b]).wait()
        pltpu.make_async_copy(x_hbm.at[0], xt_v[b], sm[b]).wait()
        pltpu.make_async_copy(geo_hbm.at[0], gt_v[b], sm[b]).wait()
        for q in range(4):
            pltpu.make_async_copy(cols_hbm[q].at[pl.ds(0, 16)],
                                  cols_v[b][q], sm[b]).wait()

    def compute(t, b):
        drain(b)
        gtv = gt_v[b][...]
        gt2 = gtv[2]
        gt3 = gtv[3]
        gt6 = gtv[6]
        gt7 = gtv[7]
        gs2 = cols_v[b][0][...]
        gs3 = cols_v[b][1][...]
        gs6 = cols_v[b][2][...]
        gs7 = cols_v[b][3][...]
        denom = gs3 + gt3 + EPS
        f1 = 2.0 * (gs6 - gt6) / denom
        f2 = 2.0 * (gs7 - gt7) / denom
        r3 = (gs2 + EPS) / (gt2 + EPS)
        r4 = (gs3 + EPS) / (gt3 + EPS)
        xtc = [xt_v[b][pl.ds(cc * 16, 16)] for cc in range(8)]
        sst = xtc[0] * xtc[0]
        for cc in range(1, 8):
            sst = sst + xtc[cc] * xtc[cc]
        tn2 = jnp.sum(sst)
        dots = jnp.zeros((16,), jnp.float32)
        sn2 = jnp.zeros((16,), jnp.float32)
        for j in range(K):
            row = xs_v[b][j, pl.ds(0, 16)]
            acc = row * xtc[0]
            acc2 = row * row
            for cc in range(1, 8):
                row = xs_v[b][j, pl.ds(cc * 16, 16)]
                acc = acc + row * xtc[cc]
                acc2 = acc2 + row * row
            dots = jnp.where(lanes == j, jnp.sum(acc), dots)
            sn2 = jnp.where(lanes == j, jnp.sum(acc2), sn2)
        off = t * K
        obuf_v[0, pl.ds(off, 16)] = f1
        obuf_v[1, pl.ds(off, 16)] = f2
        obuf_v[2, pl.ds(off, 16)] = r3
        obuf_v[3, pl.ds(off, 16)] = r4
        obuf_v[4, pl.ds(off, 16)] = dots
        obuf_v[5, pl.ds(off, 16)] = sn2 * tn2

    def body(u, carry):
        for b in range(2):
            t = 2 * u + b
            compute(t, b)
            issue(t + 2, b)
        return carry

    jax.lax.fori_loop(0, TPW // 2, body, 0)
    for b in range(2):
        drain(b)
    pltpu.sync_copy(obuf_v, out_hbm.at[wid])


def _edge_attr_sc(x, nbr_off, geometric_info):
    nbr_pad = jnp.pad(nbr_off, ((0, NTPAD - N), (0, 0)))
    geo16 = jnp.pad(geometric_info, ((0, 0), (0, 8)))
    c2 = geometric_info[:, 2]
    c3 = geometric_info[:, 3]
    c6 = geometric_info[:, 6]
    c7 = geometric_info[:, 7]
    mesh = plsc.VectorSubcoreMesh(core_axis_name="c", subcore_axis_name="s")
    f = pl.kernel(
        _edge_sc_body,
        mesh=mesh,
        compiler_params=pltpu.CompilerParams(needs_layout_passes=False),
        out_type=jax.ShapeDtypeStruct((NW, 6, EW), jnp.float32),
        scratch_types=[
            pltpu.VMEM((TPW, K), jnp.int32),
            pltpu.VMEM((16,), jnp.float32),
            pltpu.VMEM((16,), jnp.float32),
            pltpu.VMEM((16,), jnp.float32),
            pltpu.VMEM((16,), jnp.float32),
            pltpu.VMEM((16,), jnp.float32),
            pltpu.VMEM((16,), jnp.float32),
            pltpu.VMEM((16,), jnp.float32),
            pltpu.VMEM((16,), jnp.float32),
            pltpu.VMEM((16,), jnp.float32),
            pltpu.VMEM((16,), jnp.float32),
            pltpu.VMEM((D_FEAT,), jnp.float32),
            pltpu.VMEM((D_FEAT,), jnp.float32),
            pltpu.VMEM((K, D_FEAT), jnp.float32),
            pltpu.VMEM((K, D_FEAT), jnp.float32),
            pltpu.VMEM((6, EW), jnp.float32),
        ] + [pltpu.SemaphoreType.DMA] * 4,
    )
    return f(x, geo16, c2, c3, c6, c7, nbr_pad)


E_BLK2 = EW


def _enc_block2(raw_ref, W1_ref, b1_ref, g1_ref, be1_ref, W2_ref, b2_ref,
                g2_ref, be2_ref, out_ref):
    raw6 = raw_ref[...].reshape(6, E_BLK2)
    f12 = raw6[0:2, :]
    f34 = jnp.log(raw6[2:4, :])
    f5 = raw6[4:5, :] / (jnp.sqrt(raw6[5:6, :]) + EPS)
    raw = jnp.concatenate([f12, f34, f5], axis=0)
    h = jnp.dot(W1_ref[...], raw, preferred_element_type=jnp.float32)
    h = h + b1_ref[...]
    mu = jnp.mean(h, axis=0, keepdims=True)
    var = jnp.mean((h - mu) ** 2, axis=0, keepdims=True)
    h = (h - mu) / jnp.sqrt(var + LN_EPS) * g1_ref[...] + be1_ref[...]
    h = _lrelu(h)
    h = jnp.dot(W2_ref[...], h, preferred_element_type=jnp.float32)
    h = h + b2_ref[...]
    mu = jnp.mean(h, axis=0, keepdims=True)
    var = jnp.mean((h - mu) ** 2, axis=0, keepdims=True)
    h = (h - mu) / jnp.sqrt(var + LN_EPS) * g2_ref[...] + be2_ref[...]
    out_ref[...] = _lrelu(h).reshape(1, 16, E_BLK2)


def _encode2(raw, W1, b1, g1, be1, W2, b2, g2, be2):
    out = pl.pallas_call(
        _enc_block2,
        grid=(NW,),
        in_specs=[
            pl.BlockSpec((1, 6, E_BLK2), lambda w: (w, 0, 0)),
            pl.BlockSpec((32, 5), lambda w: (0, 0)),
            pl.BlockSpec((32, 1), lambda w: (0, 0)),
            pl.BlockSpec((32, 1), lambda w: (0, 0)),
            pl.BlockSpec((32, 1), lambda w: (0, 0)),
            pl.BlockSpec((16, 32), lambda w: (0, 0)),
            pl.BlockSpec((16, 1), lambda w: (0, 0)),
            pl.BlockSpec((16, 1), lambda w: (0, 0)),
            pl.BlockSpec((16, 1), lambda w: (0, 0)),
        ],
        out_specs=pl.BlockSpec((1, 16, E_BLK2), lambda w: (w, 0, 0)),
        out_shape=jax.ShapeDtypeStruct((NW, 16, EW), jnp.float32),
    )(raw, W1, b1.reshape(32, 1), g1.reshape(32, 1), be1.reshape(32, 1),
      W2, b2.reshape(16, 1), g2.reshape(16, 1), be2.reshape(16, 1))
    return out.transpose(0, 2, 1).reshape(NW * EW, 16)[:E]


def kernel(x, geometric_info, k, W1, b1, g1, be1, W2, b2, g2, be2):
    nbr = _knn_topk(geometric_info)
    off = (jnp.asarray(k) - K).astype(jnp.int32)
    nbr_off = nbr + off
    raw = _edge_attr_sc(x, nbr_off, geometric_info)
    return _encode2(raw, W1, b1, g1, be1, W2, b2, g2, be2)


# speedup vs baseline: 1.2408x; 1.2408x over previous
import functools

import jax
import jax.numpy as jnp
from jax.experimental import pallas as pl
from jax.experimental.pallas import tpu as pltpu
from jax.experimental.pallas import tpu_sc as plsc

N = 10000
D_FEAT = 128
K = 16
EPS = 1e-8
LN_EPS = 1e-5
SLOPE = 0.1

E = N * K
E_BLK = 3200


def _lrelu(h):
    return jnp.where(h >= 0, h, SLOPE * h)


def _enc_block(raw_ref, W1_ref, b1_ref, g1_ref, be1_ref, W2_ref, b2_ref,
               g2_ref, be2_ref, out_ref):
    raw = raw_ref[...]
    h = jnp.dot(raw, W1_ref[...].T, preferred_element_type=jnp.float32)
    h = h + b1_ref[...]
    mu = jnp.mean(h, axis=-1, keepdims=True)
    var = jnp.mean((h - mu) ** 2, axis=-1, keepdims=True)
    h = (h - mu) / jnp.sqrt(var + LN_EPS) * g1_ref[...] + be1_ref[...]
    h = _lrelu(h)
    h = jnp.dot(h, W2_ref[...].T, preferred_element_type=jnp.float32)
    h = h + b2_ref[...]
    mu = jnp.mean(h, axis=-1, keepdims=True)
    var = jnp.mean((h - mu) ** 2, axis=-1, keepdims=True)
    h = (h - mu) / jnp.sqrt(var + LN_EPS) * g2_ref[...] + be2_ref[...]
    out_ref[...] = _lrelu(h)


def _encode(raw, W1, b1, g1, be1, W2, b2, g2, be2):
    grid = E // E_BLK
    return pl.pallas_call(
        _enc_block,
        grid=(grid,),
        in_specs=[
            pl.BlockSpec((E_BLK, 5), lambda i: (i, 0)),
            pl.BlockSpec((32, 5), lambda i: (0, 0)),
            pl.BlockSpec((1, 32), lambda i: (0, 0)),
            pl.BlockSpec((1, 32), lambda i: (0, 0)),
            pl.BlockSpec((1, 32), lambda i: (0, 0)),
            pl.BlockSpec((16, 32), lambda i: (0, 0)),
            pl.BlockSpec((1, 16), lambda i: (0, 0)),
            pl.BlockSpec((1, 16), lambda i: (0, 0)),
            pl.BlockSpec((1, 16), lambda i: (0, 0)),
        ],
        out_specs=pl.BlockSpec((E_BLK, 16), lambda i: (i, 0)),
        out_shape=jax.ShapeDtypeStruct((E, 16), jnp.float32),
    )(raw, W1, b1.reshape(1, 32), g1.reshape(1, 32), be1.reshape(1, 32),
      W2, b2.reshape(1, 16), g2.reshape(1, 16), be2.reshape(1, 16))


NPAD = 10240
R_BLK = 128
N_BLKS = 79


F_FOLD = 16
W_FOLD = NPAD // F_FOLD
INF = 3e38


def _rbf16(v):
    u = jax.lax.bitcast_convert_type(v, jnp.int32)
    u = u + 0x7FFF + ((u >> 16) & 1)
    u = (u >> 16) << 16
    return jax.lax.bitcast_convert_type(u, jnp.float32)


def _knn_block(posr0, posr1, sqr, posc0, posc1, sqc, out_ref):
    i = pl.program_id(0)
    r0 = _rbf16(posr0[...])
    r1 = _rbf16(posr1[...])
    rs = sqr[...]
    colj = jax.lax.broadcasted_iota(jnp.int32, (R_BLK, W_FOLD), 1)
    rowg = jax.lax.broadcasted_iota(jnp.int32, (R_BLK, W_FOLD), 0) + i * R_BLK

    M1 = jnp.full((R_BLK, W_FOLD), INF, jnp.float32)
    M2 = M1
    M3 = M1
    FI1 = jnp.zeros((R_BLK, W_FOLD), jnp.int32)
    FI2 = FI1
    FI3 = FI1
    for c in range(F_FOLD):
        c0 = _rbf16(posc0[c:c + 1, :])
        c1 = _rbf16(posc1[c:c + 1, :])
        cs = sqc[c:c + 1, :]
        d = rs + cs - 2.0 * (r0 * c0 + r1 * c1)
        d = jnp.where(colj * F_FOLD + c == rowg, d + 1e10, d)
        lt1 = d < M1
        lt2 = d < M2
        lt3 = d < M3
        ci = jnp.full((R_BLK, W_FOLD), c, jnp.int32)
        M3 = jnp.where(lt3, jnp.where(lt2, M2, d), M3)
        FI3 = jnp.where(lt3, jnp.where(lt2, FI2, ci), FI3)
        M2 = jnp.where(lt2, jnp.where(lt1, M1, d), M2)
        FI2 = jnp.where(lt2, jnp.where(lt1, FI1, ci), FI2)
        M1 = jnp.where(lt1, d, M1)
        FI1 = jnp.where(lt1, ci, FI1)

    tcol = jax.lax.broadcasted_iota(jnp.int32, (R_BLK, K), 1)
    nbr = jnp.zeros((R_BLK, K), jnp.int32)
    for t in range(K):
        m = jnp.min(M1, axis=1, keepdims=True)
        hit = M1 == m
        jj = jnp.min(jnp.where(hit, colj, NPAD), axis=1, keepdims=True)
        upd = colj == jj
        fi = jnp.min(jnp.where(upd, FI1, F_FOLD), axis=1, keepdims=True)
        nbr = jnp.where(tcol == t, jj * F_FOLD + fi, nbr)
        M1 = jnp.where(upd, M2, M1)
        FI1 = jnp.where(upd, FI2, FI1)
        M2 = jnp.where(upd, M3, M2)
        FI2 = jnp.where(upd, FI3, FI2)
        M3 = jnp.where(upd, INF, M3)
    out_ref[...] = nbr


def _knn_topk(geometric_info):
    pos0 = geometric_info[:, 6]
    pos1 = geometric_info[:, 7]
    sq = pos0 * pos0 + pos1 * pos1
    pad = NPAD - N
    pos0f = jnp.pad(pos0, (0, pad)).reshape(W_FOLD, F_FOLD).T
    pos1f = jnp.pad(pos1, (0, pad)).reshape(W_FOLD, F_FOLD).T
    sqf = jnp.pad(sq, (0, pad), constant_values=3e38).reshape(W_FOLD, F_FOLD).T
    rpad = N_BLKS * R_BLK - N
    row0 = jnp.pad(pos0, (0, rpad)).reshape(-1, 1)
    row1 = jnp.pad(pos1, (0, rpad)).reshape(-1, 1)
    rowsq = jnp.pad(sq, (0, rpad)).reshape(-1, 1)
    nbr = pl.pallas_call(
        _knn_block,
        grid=(N_BLKS,),
        in_specs=[
            pl.BlockSpec((R_BLK, 1), lambda i: (i, 0)),
            pl.BlockSpec((R_BLK, 1), lambda i: (i, 0)),
            pl.BlockSpec((R_BLK, 1), lambda i: (i, 0)),
            pl.BlockSpec((F_FOLD, W_FOLD), lambda i: (0, 0)),
            pl.BlockSpec((F_FOLD, W_FOLD), lambda i: (0, 0)),
            pl.BlockSpec((F_FOLD, W_FOLD), lambda i: (0, 0)),
        ],
        out_specs=pl.BlockSpec((R_BLK, K), lambda i: (i, 0)),
        out_shape=jax.ShapeDtypeStruct((N_BLKS * R_BLK, K), jnp.int32),
    )(row0, row1, rowsq, pos0f, pos1f, sqf)
    return nbr[:N]


NW = 32
TPW = 320
NTPAD = NW * TPW
EW = TPW * K


def _norm_block(x_ref, o_ref):
    xv = x_ref[...]
    o_ref[...] = jnp.sqrt(jnp.sum(xv * xv, axis=1, keepdims=True))


def _x_norms(x):
    out = pl.pallas_call(
        _norm_block,
        grid=(25,),
        in_specs=[pl.BlockSpec((400, D_FEAT), lambda i: (i, 0))],
        out_specs=pl.BlockSpec((400, 1), lambda i: (i, 0)),
        out_shape=jax.ShapeDtypeStruct((N, 1), jnp.float32),
    )(x)
    return out.reshape(N)


def _edge_sc_body(x_hbm, geo_hbm, c2_hbm, c3_hbm, c6_hbm, c7_hbm, nbr_hbm,
                  out_hbm,
                  nbrs_v, gt0_v, gt1_v, c2b0, c2b1, c3b0, c3b1, c6b0, c6b1,
                  c7b0, c7b1, xt0_v, xt1_v, xs0_v, xs1_v, obuf_v,
                  sg0, sg1, sm0, sm1):
    cid = jax.lax.axis_index("c")
    sid = jax.lax.axis_index("s")
    wid = sid * 2 + cid
    base = wid * TPW
    pltpu.sync_copy(nbr_hbm.at[pl.ds(base, TPW)], nbrs_v)
    lanes = jax.lax.iota(jnp.int32, 16)
    xs_v = (xs0_v, xs1_v)
    xt_v = (xt0_v, xt1_v)
    gt_v = (gt0_v, gt1_v)
    cols_v = ((c2b0, c3b0, c6b0, c7b0), (c2b1, c3b1, c6b1, c7b1))
    cols_hbm = (c2_hbm, c3_hbm, c6_hbm, c7_hbm)
    sg = (sg0, sg1)
    sm = (sm0, sm1)

    def issue(t, b):
        tc = jnp.minimum(t, TPW - 1)
        i_safe = jnp.minimum(base + tc, N - 1)
        idx = nbrs_v[tc]
        pltpu.async_copy(x_hbm.at[idx], xs_v[b], sg[b])
        pltpu.async_copy(x_hbm.at[i_safe], xt_v[b], sm[b])
        pltpu.async_copy(geo_hbm.at[i_safe], gt_v[b], sm[b])
        for q in range(4):
            pltpu.async_copy(cols_hbm[q].at[idx], cols_v[b][q], sm[b])

    issue(0, 0)
    issue(1, 1)

    def drain(b):
        pltpu.make_async_copy(x_hbm.at[pl.ds(0, 16)], xs_v[b], sg[b]).wait()
        pltpu.make_async_copy(x_hbm.at[0], xt_v[b], sm[b]).wait()
        pltpu.make_async_copy(geo_hbm.at[0], gt_v[b], sm[b]).wait()
        for q in range(4):
            pltpu.make_async_copy(cols_hbm[q].at[pl.ds(0, 16)],
                                  cols_v[b][q], sm[b]).wait()

    def compute(t, b):
        drain(b)
        gtv = gt_v[b][...]
        gt2 = gtv[2]
        gt3 = gtv[3]
        gt6 = gtv[6]
        gt7 = gtv[7]
        gs2 = cols_v[b][0][...]
        gs3 = cols_v[b][1][...]
        gs6 = cols_v[b][2][...]
        gs7 = cols_v[b][3][...]
        denom = gs3 + gt3 + EPS
        f1 = 2.0 * (gs6 - gt6) / denom
        f2 = 2.0 * (gs7 - gt7) / denom
        r3 = (gs2 + EPS) / (gt2 + EPS)
        r4 = (gs3 + EPS) / (gt3 + EPS)
        xtc = [xt_v[b][pl.ds(cc * 16, 16)] for cc in range(8)]
        sst = xtc[0] * xtc[0]
        for cc in range(1, 8):
            sst = sst + xtc[cc] * xtc[cc]
        tn2 = jnp.sum(sst)
        dots = jnp.zeros((16,), jnp.float32)
        sn2 = jnp.zeros((16,), jnp.float32)
        for j in range(K):
            row = xs_v[b][j, pl.ds(0, 16)]
            acc = row * xtc[0]
            acc2 = row * row
            for cc in range(1, 8):
                row = xs_v[b][j, pl.ds(cc * 16, 16)]
                acc = acc + row * xtc[cc]
                acc2 = acc2 + row * row
            dots = jnp.where(lanes == j, jnp.sum(acc), dots)
            sn2 = jnp.where(lanes == j, jnp.sum(acc2), sn2)
        off = t * K
        obuf_v[0, pl.ds(off, 16)] = f1
        obuf_v[1, pl.ds(off, 16)] = f2
        obuf_v[2, pl.ds(off, 16)] = r3
        obuf_v[3, pl.ds(off, 16)] = r4
        obuf_v[4, pl.ds(off, 16)] = dots
        obuf_v[5, pl.ds(off, 16)] = sn2 * tn2

    def body(u, carry):
        for b in range(2):
            t = 2 * u + b
            compute(t, b)
            issue(t + 2, b)
        return carry

    jax.lax.fori_loop(0, TPW // 2, body, 0)
    for b in range(2):
        drain(b)
    pltpu.sync_copy(obuf_v, out_hbm.at[wid])


def _edge_attr_sc(x, nbr_off, geometric_info):
    nbr_pad = jnp.pad(nbr_off, ((0, NTPAD - N), (0, 0)))
    geo16 = jnp.pad(geometric_info, ((0, 0), (0, 8)))
    c2 = geometric_info[:, 2]
    c3 = geometric_info[:, 3]
    c6 = geometric_info[:, 6]
    c7 = geometric_info[:, 7]
    mesh = plsc.VectorSubcoreMesh(core_axis_name="c", subcore_axis_name="s")
    f = pl.kernel(
        _edge_sc_body,
        mesh=mesh,
        compiler_params=pltpu.CompilerParams(needs_layout_passes=False),
        out_type=jax.ShapeDtypeStruct((NW, 6, EW), jnp.float32),
        scratch_types=[
            pltpu.VMEM((TPW, K), jnp.int32),
            pltpu.VMEM((16,), jnp.float32),
            pltpu.VMEM((16,), jnp.float32),
            pltpu.VMEM((16,), jnp.float32),
            pltpu.VMEM((16,), jnp.float32),
            pltpu.VMEM((16,), jnp.float32),
            pltpu.VMEM((16,), jnp.float32),
            pltpu.VMEM((16,), jnp.float32),
            pltpu.VMEM((16,), jnp.float32),
            pltpu.VMEM((16,), jnp.float32),
            pltpu.VMEM((16,), jnp.float32),
            pltpu.VMEM((D_FEAT,), jnp.float32),
            pltpu.VMEM((D_FEAT,), jnp.float32),
            pltpu.VMEM((K, D_FEAT), jnp.float32),
            pltpu.VMEM((K, D_FEAT), jnp.float32),
            pltpu.VMEM((6, EW), jnp.float32),
        ] + [pltpu.SemaphoreType.DMA] * 4,
    )
    return f(x, geo16, c2, c3, c6, c7, nbr_pad)


E_BLK2 = EW


def _enc_block2(raw_ref, W1_ref, b1_ref, g1_ref, be1_ref, W2_ref, b2_ref,
                g2_ref, be2_ref, out_ref):
    raw6 = raw_ref[...].reshape(6, E_BLK2)
    f12 = raw6[0:2, :]
    f34 = jnp.log(raw6[2:4, :])
    f5 = raw6[4:5, :] / (jnp.sqrt(raw6[5:6, :]) + EPS)
    raw = jnp.concatenate([f12, f34, f5], axis=0)
    h = jnp.dot(W1_ref[...], raw, preferred_element_type=jnp.float32)
    h = h + b1_ref[...]
    mu = jnp.mean(h, axis=0, keepdims=True)
    var = jnp.mean((h - mu) ** 2, axis=0, keepdims=True)
    h = (h - mu) / jnp.sqrt(var + LN_EPS) * g1_ref[...] + be1_ref[...]
    h = _lrelu(h)
    h = jnp.dot(W2_ref[...], h, preferred_element_type=jnp.float32)
    h = h + b2_ref[...]
    mu = jnp.mean(h, axis=0, keepdims=True)
    var = jnp.mean((h - mu) ** 2, axis=0, keepdims=True)
    h = (h - mu) / jnp.sqrt(var + LN_EPS) * g2_ref[...] + be2_ref[...]
    out_ref[...] = _lrelu(h).reshape(1, 16, E_BLK2)


def _encode2(raw, W1, b1, g1, be1, W2, b2, g2, be2):
    out = pl.pallas_call(
        _enc_block2,
        grid=(NW,),
        in_specs=[
            pl.BlockSpec((1, 6, E_BLK2), lambda w: (w, 0, 0)),
            pl.BlockSpec((32, 5), lambda w: (0, 0)),
            pl.BlockSpec((32, 1), lambda w: (0, 0)),
            pl.BlockSpec((32, 1), lambda w: (0, 0)),
            pl.BlockSpec((32, 1), lambda w: (0, 0)),
            pl.BlockSpec((16, 32), lambda w: (0, 0)),
            pl.BlockSpec((16, 1), lambda w: (0, 0)),
            pl.BlockSpec((16, 1), lambda w: (0, 0)),
            pl.BlockSpec((16, 1), lambda w: (0, 0)),
        ],
        out_specs=pl.BlockSpec((1, 16, E_BLK2), lambda w: (w, 0, 0)),
        out_shape=jax.ShapeDtypeStruct((NW, 16, EW), jnp.float32),
    )(raw, W1, b1.reshape(32, 1), g1.reshape(32, 1), be1.reshape(32, 1),
      W2, b2.reshape(16, 1), g2.reshape(16, 1), be2.reshape(16, 1))
    return out.transpose(0, 2, 1).reshape(NW * EW, 16)[:E]


def kernel(x, geometric_info, k, W1, b1, g1, be1, W2, b2, g2, be2):
    nbr = _knn_topk(geometric_info)
    off = (jnp.asarray(k) - K).astype(jnp.int32)
    nbr_off = nbr + off
    raw = _edge_attr_sc(x, nbr_off, geometric_info)
    return _encode2(raw, W1, b1, g1, be1, W2, b2, g2, be2)
